# Initial kernel scaffold; baseline (speedup 1.0000x reference)
import jax
import jax.numpy as jnp
from jax.experimental import pallas as pl


def kernel(x, edge_index, edge_attrs, W):
    raise NotImplementedError("write your pallas kernel here")


# speedup vs baseline: 1.4255x; 1.4255x over previous
import functools

import jax
import jax.numpy as jnp
import numpy as np
from jax import lax
from jax.experimental import pallas as pl
from jax.experimental.pallas import tpu as pltpu
from jax.experimental.pallas import tpu_sc as plsc

N = 10000
D = 128
A = 4
E = 320000

NC = 2
NS = 16
NW = NC * NS
L = 16

B = 48
EPT = 10032
NCHUNK = EPT // B
E_PAD = EPT * NW
N_PAD = 10240
JUNK = 10016
R = D + L
ALPHA = 1.0 / np.sqrt(D * A)


def _premul_body(x_ref, w_ref, u_ref):
    u_ref[...] = jnp.dot(x_ref[...], w_ref[...],
                         preferred_element_type=jnp.float32) * ALPHA


def _premul(x, w2):
    return pl.pallas_call(
        _premul_body,
        grid=(10,),
        in_specs=[
            pl.BlockSpec((1000, D), lambda i: (i, 0)),
            pl.BlockSpec((D, A * D), lambda i: (0, 0)),
        ],
        out_specs=pl.BlockSpec((1000, A * D), lambda i: (i, 0)),
        out_shape=jax.ShapeDtypeStruct((N, A * D), jnp.float32),
    )(x, w2)


def _sc_body(u_hbm, src_hbm, dst_hbm, attrs_hbm, out_hbm,
             acc_sh, src_v, dst_v, attrs_v, rows_v, msg_v, sem):
    cid = lax.axis_index("c")
    sid = lax.axis_index("s")
    tid = cid * NS + sid

    rows_per_tile = N_PAD // NS

    zvec = jnp.zeros((L,), jnp.float32)

    def _zrow(b, _):
        for q in range(R // L):
            msg_v[b, pl.ds(q * L, L)] = zvec
        return 0

    lax.fori_loop(0, B, _zrow, 0)
    nfull = rows_per_tile // B
    rem = rows_per_tile - nfull * B
    for k in range(nfull):
        pltpu.sync_copy(msg_v, acc_sh.at[pl.ds(sid * rows_per_tile + k * B, B), :])
    if rem:
        pltpu.sync_copy(
            msg_v.at[pl.ds(0, rem), :],
            acc_sh.at[pl.ds(sid * rows_per_tile + nfull * B, rem), :])

    e0 = jnp.where(lax.iota(jnp.int32, L) == 0, 1.0, 0.0).astype(jnp.float32)

    def _erow(b, _):
        msg_v[b, pl.ds(D, L)] = e0
        return 0

    lax.fori_loop(0, B, _erow, 0)

    plsc.subcore_barrier()

    def _chunk(c, _):
        base = tid * EPT + c * B
        pltpu.sync_copy(src_hbm.at[pl.ds(base, B)], src_v)
        pltpu.sync_copy(dst_hbm.at[pl.ds(base, B)], dst_v)
        pltpu.sync_copy(attrs_hbm.at[pl.ds(base * A, B * A)], attrs_v)
        pltpu.async_copy(u_hbm.at[src_v], rows_v, sem).wait()

        def _edge4(g, _):
            avec = attrs_v[pl.ds(g * (4 * A), 4 * A)]
            for k in range(4):
                b = 4 * g + k
                a0 = avec[k * A + 0]
                a1 = avec[k * A + 1]
                a2 = avec[k * A + 2]
                a3 = avec[k * A + 3]
                for q in range(D // L):
                    v = (a0 * rows_v[b, pl.ds(q * L, L)]
                         + a1 * rows_v[b, pl.ds(D + q * L, L)]
                         + a2 * rows_v[b, pl.ds(2 * D + q * L, L)]
                         + a3 * rows_v[b, pl.ds(3 * D + q * L, L)])
                    msg_v[b, pl.ds(q * L, L)] = v
            return 0

        lax.fori_loop(0, B // 4, _edge4, 0)
        pltpu.sync_copy(msg_v, acc_sh.at[dst_v], add=True)
        return 0

    lax.fori_loop(0, NCHUNK, _chunk, 0)

    plsc.subcore_barrier()

    pltpu.sync_copy(acc_sh.at[pl.ds(sid * rows_per_tile, rows_per_tile), :],
                    out_hbm.at[cid, pl.ds(sid * rows_per_tile, rows_per_tile), :])


def _sc_message_pass(u, src_pad, dst_pad, attrs_pad):
    mesh = plsc.VectorSubcoreMesh(core_axis_name="c", subcore_axis_name="s")
    kfn = pl.kernel(
        _sc_body,
        out_type=jax.ShapeDtypeStruct((NC, N_PAD, R), jnp.float32),
        mesh=mesh,
        scratch_types=[
            pltpu.VMEM_SHARED((N_PAD, R), jnp.float32),
            pltpu.VMEM((B,), jnp.int32),
            pltpu.VMEM((B,), jnp.int32),
            pltpu.VMEM((B * A,), jnp.float32),
            pltpu.VMEM((B, A * D), jnp.float32),
            pltpu.VMEM((B, R), jnp.float32),
            pltpu.SemaphoreType.DMA,
        ],
        compiler_params=pltpu.CompilerParams(use_tc_tiling_on_sc=False),
    )
    return kfn(u, src_pad, dst_pad, attrs_pad)


def _fin_body(p_ref, o_ref):
    s = p_ref[0] + p_ref[1]
    cnt = s[:, D:D + 1]
    o_ref[...] = s[:, :D] / jnp.maximum(cnt, 1.0)


def _finalize(partials):
    return pl.pallas_call(
        _fin_body,
        grid=(10,),
        in_specs=[pl.BlockSpec((NC, 1000, R), lambda i: (0, i, 0))],
        out_specs=pl.BlockSpec((1000, D), lambda i: (i, 0)),
        out_shape=jax.ShapeDtypeStruct((N, D), jnp.float32),
    )(partials)


@jax.jit
def kernel(x, edge_index, edge_attrs, W):
    w2 = W.reshape(D, A * D)
    u = _premul(x, w2)

    src = edge_index[0]
    dst = edge_index[1]
    pad = E_PAD - E
    src_pad = jnp.concatenate([src, jnp.zeros((pad,), jnp.int32)])
    dst_pad = jnp.concatenate([dst, jnp.full((pad,), JUNK, jnp.int32)])
    attrs_pad = jnp.concatenate(
        [edge_attrs, jnp.zeros((pad, A), jnp.float32)], axis=0).reshape(-1)

    partials = _sc_message_pass(u, src_pad, dst_pad, attrs_pad)
    return _finalize(partials)


# speedup vs baseline: 3.5489x; 2.4895x over previous
import functools

import jax
import jax.numpy as jnp
import numpy as np
from jax import lax
from jax.experimental import pallas as pl
from jax.experimental.pallas import tpu as pltpu
from jax.experimental.pallas import tpu_sc as plsc

N = 10000
D = 128
A = 4
E = 320000

NC = 2
NS = 16
NW = NC * NS
L = 16

B = 40
CPS = 10
NSUP = 25
EPT = B * CPS * NSUP
R = D + L
RPT = N // NS
ALPHA = 1.0 / np.sqrt(D * A)

_PERM = np.empty((A * D,), np.int32)
for _q in range(A * D // 32):
    for _t in range(16):
        _PERM[32 * _q + 2 * _t] = 32 * _q + _t
        _PERM[32 * _q + 2 * _t + 1] = 32 * _q + 16 + _t


def _premul_body(x_ref, w_ref, u_ref):
    u_ref[...] = (jnp.dot(x_ref[...], w_ref[...],
                          preferred_element_type=jnp.float32)
                  * ALPHA).astype(jnp.bfloat16)


def _premul(x, w2p):
    return pl.pallas_call(
        _premul_body,
        grid=(5,),
        in_specs=[
            pl.BlockSpec((2000, D), lambda i: (i, 0)),
            pl.BlockSpec((D, A * D), lambda i: (0, 0)),
        ],
        out_specs=pl.BlockSpec((2000, A * D), lambda i: (i, 0)),
        out_shape=jax.ShapeDtypeStruct((N, A * D), jnp.bfloat16),
    )(x, w2p)


def _sc_body(u_hbm, src_hbm, dst_hbm, attrs_hbm, out_hbm,
             acc_sh, src_v, dst_v, attrs_v, r0, r1, msg_v, s0, s1):
    cid = lax.axis_index("c")
    sid = lax.axis_index("s")
    tid = cid * NS + sid

    zvec = jnp.zeros((L,), jnp.float32)

    def _zrow(b, _):
        for q in range(R // L):
            msg_v[b, pl.ds(q * L, L)] = zvec
        return 0

    lax.fori_loop(0, B, _zrow, 0)
    nfull = RPT // B
    rem = RPT - nfull * B
    for k in range(nfull):
        pltpu.sync_copy(msg_v, acc_sh.at[pl.ds(sid * RPT + k * B, B), :])
    if rem:
        pltpu.sync_copy(msg_v.at[pl.ds(0, rem), :],
                        acc_sh.at[pl.ds(sid * RPT + nfull * B, rem), :])

    e0 = jnp.where(lax.iota(jnp.int32, L) == 0, 1.0, 0.0).astype(jnp.float32)

    def _erow(b, _):
        msg_v[b, pl.ds(D, L)] = e0
        return 0

    lax.fori_loop(0, B, _erow, 0)

    plsc.subcore_barrier()

    def _super(s, _):
        row0 = tid * (EPT // B) + s * CPS
        pltpu.sync_copy(src_hbm.at[pl.ds(row0, CPS), :], src_v)
        pltpu.sync_copy(dst_hbm.at[pl.ds(row0, CPS), :], dst_v)
        pltpu.sync_copy(attrs_hbm.at[pl.ds(row0 * B * A, CPS * B * A)],
                        attrs_v)

        bufs = (r0, r1)
        sems = (s0, s1)
        descs = [None] * CPS
        descs[0] = pltpu.async_copy(u_hbm.at[src_v.at[0]], r0, s0)
        for c in range(CPS):
            cur = bufs[c % 2]
            if c + 1 < CPS:
                descs[c + 1] = pltpu.async_copy(
                    u_hbm.at[src_v.at[c + 1]], bufs[(c + 1) % 2],
                    sems[(c + 1) % 2])
            descs[c].wait()

            def _grp(g, _, c=c, cur=cur):
                avec = attrs_v[pl.ds((c * B + 4 * g) * A, 4 * A)]
                for k in range(4):
                    b = 4 * g + k
                    acc = [None] * (D // L)
                    for j in range(A):
                        aj = avec[k * A + j]
                        for p in range(D // 32):
                            va, vb = plsc.unpack(
                                cur[b, pl.ds(j * D + 32 * p, 32)],
                                format=plsc.PackFormat.INTERLEAVED)
                            if j == 0:
                                acc[2 * p] = aj * va
                                acc[2 * p + 1] = aj * vb
                            else:
                                acc[2 * p] += aj * va
                                acc[2 * p + 1] += aj * vb
                    for t in range(D // L):
                        msg_v[b, pl.ds(t * L, L)] = acc[t]
                return 0

            lax.fori_loop(0, B // 4, _grp, 0)
            pltpu.sync_copy(msg_v, acc_sh.at[dst_v.at[c]], add=True)
        return 0

    lax.fori_loop(0, NSUP, _super, 0)

    plsc.subcore_barrier()

    pltpu.sync_copy(acc_sh.at[pl.ds(sid * RPT, RPT), :],
                    out_hbm.at[cid, pl.ds(sid * RPT, RPT), :])


def _sc_message_pass(u, src2, dst2, attrs_flat):
    mesh = plsc.VectorSubcoreMesh(core_axis_name="c", subcore_axis_name="s")
    kfn = pl.kernel(
        _sc_body,
        out_type=jax.ShapeDtypeStruct((NC, N, R), jnp.float32),
        mesh=mesh,
        scratch_types=[
            pltpu.VMEM_SHARED((N, R), jnp.float32),
            pltpu.VMEM((CPS, B), jnp.int32),
            pltpu.VMEM((CPS, B), jnp.int32),
            pltpu.VMEM((CPS * B * A,), jnp.float32),
            pltpu.VMEM((B, A * D), jnp.bfloat16),
            pltpu.VMEM((B, A * D), jnp.bfloat16),
            pltpu.VMEM((B, R), jnp.float32),
            pltpu.SemaphoreType.DMA,
            pltpu.SemaphoreType.DMA,
        ],
        compiler_params=pltpu.CompilerParams(use_tc_tiling_on_sc=False,
                                             needs_layout_passes=False),
    )
    return kfn(u, src2, dst2, attrs_flat)


def _fin_body(p_ref, o_ref):
    s = p_ref[0] + p_ref[1]
    cnt = s[:, D:D + 1]
    o_ref[...] = s[:, :D] / jnp.maximum(cnt, 1.0)


def _finalize(partials):
    return pl.pallas_call(
        _fin_body,
        grid=(10,),
        in_specs=[pl.BlockSpec((NC, 1000, R), lambda i: (0, i, 0))],
        out_specs=pl.BlockSpec((1000, D), lambda i: (i, 0)),
        out_shape=jax.ShapeDtypeStruct((N, D), jnp.float32),
    )(partials)


@jax.jit
def kernel(x, edge_index, edge_attrs, W):
    w2p = W.reshape(D, A * D)[:, _PERM]
    u = _premul(x, w2p)

    src2 = edge_index[0].reshape(E // B, B)
    dst2 = edge_index[1].reshape(E // B, B)
    attrs_flat = edge_attrs.reshape(-1)

    partials = _sc_message_pass(u, src2, dst2, attrs_flat)
    return _finalize(partials)
